# Initial kernel scaffold; baseline (speedup 1.0000x reference)
import jax
import jax.numpy as jnp
from jax import lax
from jax.experimental import pallas as pl
from jax.experimental.pallas import tpu as pltpu
from jax.experimental.pallas import tpu_sc as plsc

NC = 2
NS = 16
NW = NC * NS
CH = 100
CNT_W = 16
ZR = 125


def _make_sc_agg(n, d, e, with_counts):
  k_chunks = e // (NW * CH)
  assert k_chunks * NW * CH == e
  rows_per_sub = n // NS
  assert rows_per_sub * NS == n and rows_per_sub % ZR == 0
  n_dma = rows_per_sub // ZR

  mesh = plsc.VectorSubcoreMesh(core_axis_name="c", subcore_axis_name="s")
  out_type = [jax.ShapeDtypeStruct((NC, n, d), jnp.float32)]
  scratch = [
      pltpu.VMEM((k_chunks, CH), jnp.int32),
      pltpu.VMEM((k_chunks, CH), jnp.int32),
      pltpu.VMEM((CH, d), jnp.float32),
      pltpu.VMEM((ZR, d), jnp.float32),
      pltpu.VMEM_SHARED((n, d), jnp.float32),
      pltpu.SemaphoreType.DMA,
  ]
  if with_counts:
    out_type.append(jax.ShapeDtypeStruct((NC, n, CNT_W), jnp.float32))
    scratch += [
        pltpu.VMEM((CH, CNT_W), jnp.float32),
        pltpu.VMEM((ZR, CNT_W), jnp.float32),
        pltpu.VMEM_SHARED((n, CNT_W), jnp.float32),
    ]

  def body(h_hbm, src_hbm, dst_hbm, *rest):
    if with_counts:
      (out_hbm, cnt_hbm, src_v, dst_v, rows_v, zrow_v, acc_sh, sem,
       ones_v, zcnt_v, cnt_sh) = rest
    else:
      (out_hbm, src_v, dst_v, rows_v, zrow_v, acc_sh, sem) = rest

    cid = lax.axis_index("c")
    sid = lax.axis_index("s")
    wid = cid * NS + sid
    base = sid * rows_per_sub

    @pl.loop(0, ZR)
    def _(r):
      @pl.loop(0, d, step=16)
      def _(c):
        zrow_v[pl.ds(r, 1), pl.ds(c, 16)] = jnp.zeros((1, 16), jnp.float32)

    if with_counts:
      @pl.loop(0, ZR)
      def _(r):
        zcnt_v[pl.ds(r, 1), pl.ds(0, 16)] = jnp.zeros((1, 16), jnp.float32)

      @pl.loop(0, CH)
      def _(r):
        ones_v[pl.ds(r, 1), pl.ds(0, 16)] = jnp.ones((1, 16), jnp.float32)

    for t in range(n_dma):
      pltpu.sync_copy(zrow_v, acc_sh.at[pl.ds(base + t * ZR, ZR)])
    if with_counts:
      for t in range(n_dma):
        pltpu.sync_copy(zcnt_v, cnt_sh.at[pl.ds(base + t * ZR, ZR)])
    plsc.subcore_barrier()

    pltpu.sync_copy(src_hbm.at[wid], src_v)
    pltpu.sync_copy(dst_hbm.at[wid], dst_v)

    @pl.loop(0, k_chunks)
    def _(j):
      pltpu.async_copy(h_hbm.at[src_v.at[j]], rows_v, sem).wait()
      pltpu.sync_copy(rows_v, acc_sh.at[dst_v.at[j]], add=True)
      if with_counts:
        pltpu.sync_copy(ones_v, cnt_sh.at[dst_v.at[j]], add=True)

    plsc.subcore_barrier()

    for t in range(n_dma):
      sl = pl.ds(base + t * ZR, ZR)
      pltpu.sync_copy(acc_sh.at[sl], out_hbm.at[cid].at[sl])
    if with_counts:
      for t in range(n_dma):
        sl = pl.ds(base + t * ZR, ZR)
        pltpu.sync_copy(cnt_sh.at[sl], cnt_hbm.at[cid].at[sl])

  return pl.kernel(
      body,
      out_type=tuple(out_type) if with_counts else out_type[0],
      mesh=mesh,
      scratch_types=scratch,
  )


def _tc_layer_body(p_ref, c_ref, h_ref, wl_ref, bl_ref, wr_ref, o_ref):
  s = p_ref[0] + p_ref[1]
  c = c_ref[0, :, 0:1] + c_ref[1, :, 0:1]
  mean = s / jnp.maximum(c, 1.0)
  z = jnp.dot(mean, wl_ref[...], preferred_element_type=jnp.float32)
  z = z + bl_ref[...]
  z = z + jnp.dot(h_ref[...], wr_ref[...], preferred_element_type=jnp.float32)
  o_ref[...] = jnp.maximum(z, 0.0)


def _tc_final_body(p_ref, c_ref, h_ref, wl_ref, bl_ref, wr_ref, wo_ref,
                   bo_ref, o_ref):
  s = p_ref[0] + p_ref[1]
  c = c_ref[0, :, 0:1] + c_ref[1, :, 0:1]
  mean = s / jnp.maximum(c, 1.0)
  z = jnp.dot(mean, wl_ref[...], preferred_element_type=jnp.float32)
  z = z + bl_ref[...]
  z = z + jnp.dot(h_ref[...], wr_ref[...], preferred_element_type=jnp.float32)
  h3 = jnp.maximum(z, 0.0)
  o_ref[...] = jnp.dot(h3, wo_ref[...],
                       preferred_element_type=jnp.float32) + bo_ref[...]


def _tc_layer(p, cnt, h, wlT, bl, wrT, blk=1000):
  n, d = h.shape
  return pl.pallas_call(
      _tc_layer_body,
      grid=(n // blk,),
      in_specs=[
          pl.BlockSpec((NC, blk, d), lambda i: (0, i, 0)),
          pl.BlockSpec((NC, blk, CNT_W), lambda i: (0, i, 0)),
          pl.BlockSpec((blk, d), lambda i: (i, 0)),
          pl.BlockSpec((d, d), lambda i: (0, 0)),
          pl.BlockSpec((1, d), lambda i: (0, 0)),
          pl.BlockSpec((d, d), lambda i: (0, 0)),
      ],
      out_specs=pl.BlockSpec((blk, d), lambda i: (i, 0)),
      out_shape=jax.ShapeDtypeStruct((n, d), jnp.float32),
  )(p, cnt, h, wlT, bl, wrT)


def _tc_final(p, cnt, h, wlT, bl, wrT, woT, bo, blk=1000):
  n, d = h.shape
  out = woT.shape[1]
  return pl.pallas_call(
      _tc_final_body,
      grid=(n // blk,),
      in_specs=[
          pl.BlockSpec((NC, blk, d), lambda i: (0, i, 0)),
          pl.BlockSpec((NC, blk, CNT_W), lambda i: (0, i, 0)),
          pl.BlockSpec((blk, d), lambda i: (i, 0)),
          pl.BlockSpec((d, d), lambda i: (0, 0)),
          pl.BlockSpec((1, d), lambda i: (0, 0)),
          pl.BlockSpec((d, d), lambda i: (0, 0)),
          pl.BlockSpec((d, out), lambda i: (0, 0)),
          pl.BlockSpec((1, out), lambda i: (0, 0)),
      ],
      out_specs=pl.BlockSpec((blk, out), lambda i: (i, 0)),
      out_shape=jax.ShapeDtypeStruct((n, out), jnp.float32),
  )(p, cnt, h, wlT, bl, wrT, woT, bo)


def kernel(x, edge_index, W1l, b1l, W1r, W2l, b2l, W2r, W3l, b3l, W3r,
           Wout, bout):
  n, d = x.shape
  e = edge_index.shape[1]
  src3 = edge_index[0].reshape(NW, -1, CH)
  dst3 = edge_index[1].reshape(NW, -1, CH)

  agg_first = _make_sc_agg(n, d, e, with_counts=True)
  agg = _make_sc_agg(n, d, e, with_counts=False)

  p1, c1 = agg_first(x, src3, dst3)
  h1 = _tc_layer(p1, c1, x, W1l.T, b1l.reshape(1, -1), W1r.T)
  p2 = agg(h1, src3, dst3)
  h2 = _tc_layer(p2, c1, h1, W2l.T, b2l.reshape(1, -1), W2r.T)
  p3 = agg(h2, src3, dst3)
  return _tc_final(p3, c1, h2, W3l.T, b3l.reshape(1, -1), W3r.T,
                   Wout.T, bout.reshape(1, -1))


# speedup vs baseline: 7.8576x; 7.8576x over previous
import jax
import jax.numpy as jnp
from jax import lax
from jax.experimental import pallas as pl
from jax.experimental.pallas import tpu as pltpu
from jax.experimental.pallas import tpu_sc as plsc

NC = 2
NS = 16
NW = NC * NS
CH = 125
CNT_W = 128
ZR = 128


def _make_sc_agg(n, d, e):
  k_chunks = e // (NW * CH)
  assert k_chunks * NW * CH == e
  rows_per_sub = n // NS
  assert rows_per_sub * NS == n and rows_per_sub % ZR == 0
  n_dma = rows_per_sub // ZR

  ib = 16
  assert k_chunks % ib == 0
  mesh = plsc.VectorSubcoreMesh(core_axis_name="c", subcore_axis_name="s")
  scratch = [
      pltpu.VMEM((ib, CH), jnp.int32),
      pltpu.VMEM((ib, CH), jnp.int32),
      pltpu.VMEM((CH, d), jnp.float32),
      pltpu.VMEM_SHARED((n, d), jnp.float32),
      pltpu.SemaphoreType.DMA,
  ]

  def body(h_hbm, src_hbm, dst_hbm, out_hbm, src_v, dst_v, rows_v,
           acc_sh, sem):
    cid = lax.axis_index("c")
    sid = lax.axis_index("s")
    wid = cid * NS + sid
    base = sid * rows_per_sub

    @pl.loop(0, CH)
    def _(r):
      @pl.loop(0, d, step=16)
      def _(c):
        rows_v[pl.ds(r, 1), pl.ds(c, 16)] = jnp.zeros((1, 16), jnp.float32)

    nz_full, nz_rem = rows_per_sub // CH, rows_per_sub % CH
    for t in range(nz_full):
      pltpu.sync_copy(rows_v, acc_sh.at[pl.ds(base + t * CH, CH)])
    if nz_rem:
      pltpu.sync_copy(rows_v.at[pl.ds(0, nz_rem)],
                      acc_sh.at[pl.ds(base + nz_full * CH, nz_rem)])
    plsc.subcore_barrier()

    @pl.loop(0, k_chunks // ib)
    def _(g):
      pltpu.sync_copy(src_hbm.at[wid].at[pl.ds(g * ib, ib)], src_v)
      pltpu.sync_copy(dst_hbm.at[wid].at[pl.ds(g * ib, ib)], dst_v)

      @pl.loop(0, ib)
      def _(j):
        pltpu.async_copy(h_hbm.at[src_v.at[j]], rows_v, sem).wait()
        pltpu.sync_copy(rows_v, acc_sh.at[dst_v.at[j]], add=True)

    plsc.subcore_barrier()

    for t in range(n_dma):
      sl = pl.ds(base + t * ZR, ZR)
      pltpu.sync_copy(acc_sh.at[sl], out_hbm.at[cid].at[sl])

  return pl.kernel(
      body,
      out_type=jax.ShapeDtypeStruct((NC, n, d), jnp.float32),
      mesh=mesh,
      scratch_types=scratch,
  )


def _make_sc_counts(n, e, cw=CNT_W):
  k_chunks = e // (NW * CH)
  rows_per_sub = n // NS
  n_dma = rows_per_sub // ZR

  mesh = plsc.VectorSubcoreMesh(core_axis_name="c", subcore_axis_name="s")
  scratch = [
      pltpu.VMEM((k_chunks, CH), jnp.int32),
      pltpu.VMEM((CH, cw), jnp.float32),
      pltpu.VMEM((ZR, cw), jnp.float32),
      pltpu.VMEM_SHARED((n, cw), jnp.float32),
  ]

  def body(dst_hbm, cnt_hbm, dst_v, ones_v, zcnt_v, cnt_sh):
    cid = lax.axis_index("c")
    sid = lax.axis_index("s")
    wid = cid * NS + sid
    base = sid * rows_per_sub

    @pl.loop(0, ZR)
    def _(r):
      @pl.loop(0, cw, step=16)
      def _(c):
        zcnt_v[pl.ds(r, 1), pl.ds(c, 16)] = jnp.zeros((1, 16), jnp.float32)

    @pl.loop(0, CH)
    def _(r):
      @pl.loop(0, cw, step=16)
      def _(c):
        ones_v[pl.ds(r, 1), pl.ds(c, 16)] = jnp.ones((1, 16), jnp.float32)

    for t in range(n_dma):
      pltpu.sync_copy(zcnt_v, cnt_sh.at[pl.ds(base + t * ZR, ZR)])
    plsc.subcore_barrier()

    pltpu.sync_copy(dst_hbm.at[wid], dst_v)

    @pl.loop(0, k_chunks)
    def _(j):
      pltpu.sync_copy(ones_v, cnt_sh.at[dst_v.at[j]], add=True)

    plsc.subcore_barrier()

    for t in range(n_dma):
      sl = pl.ds(base + t * ZR, ZR)
      pltpu.sync_copy(cnt_sh.at[sl], cnt_hbm.at[cid].at[sl])

  return pl.kernel(
      body,
      out_type=jax.ShapeDtypeStruct((NC, n, cw), jnp.float32),
      mesh=mesh,
      scratch_types=scratch,
  )


def _tc_layer_body(p_ref, c_ref, h_ref, wl_ref, bl_ref, wr_ref, o_ref):
  s = p_ref[0] + p_ref[1]
  c = c_ref[0, :, 0:1] + c_ref[1, :, 0:1]
  mean = s / jnp.maximum(c, 1.0)
  z = jnp.dot(mean, wl_ref[...], preferred_element_type=jnp.float32)
  z = z + bl_ref[...]
  z = z + jnp.dot(h_ref[...], wr_ref[...], preferred_element_type=jnp.float32)
  o_ref[...] = jnp.maximum(z, 0.0)


def _tc_final_body(p_ref, c_ref, h_ref, wl_ref, bl_ref, wr_ref, wo_ref,
                   bo_ref, o_ref):
  s = p_ref[0] + p_ref[1]
  c = c_ref[0, :, 0:1] + c_ref[1, :, 0:1]
  mean = s / jnp.maximum(c, 1.0)
  z = jnp.dot(mean, wl_ref[...], preferred_element_type=jnp.float32)
  z = z + bl_ref[...]
  z = z + jnp.dot(h_ref[...], wr_ref[...], preferred_element_type=jnp.float32)
  h3 = jnp.maximum(z, 0.0)
  o_ref[...] = jnp.dot(h3, wo_ref[...],
                       preferred_element_type=jnp.float32) + bo_ref[...]


def _tc_layer(p, cnt, h, wlT, bl, wrT, blk=1024):
  n, d = h.shape
  return pl.pallas_call(
      _tc_layer_body,
      grid=(n // blk,),
      in_specs=[
          pl.BlockSpec((NC, blk, d), lambda i: (0, i, 0)),
          pl.BlockSpec((NC, blk, CNT_W), lambda i: (0, i, 0)),
          pl.BlockSpec((blk, d), lambda i: (i, 0)),
          pl.BlockSpec((d, d), lambda i: (0, 0)),
          pl.BlockSpec((1, d), lambda i: (0, 0)),
          pl.BlockSpec((d, d), lambda i: (0, 0)),
      ],
      out_specs=pl.BlockSpec((blk, d), lambda i: (i, 0)),
      out_shape=jax.ShapeDtypeStruct((n, d), jnp.float32),
  )(p, cnt, h, wlT, bl, wrT)


def _tc_final(p, cnt, h, wlT, bl, wrT, woT, bo, blk=1024):
  n, d = h.shape
  out = woT.shape[1]
  return pl.pallas_call(
      _tc_final_body,
      grid=(n // blk,),
      in_specs=[
          pl.BlockSpec((NC, blk, d), lambda i: (0, i, 0)),
          pl.BlockSpec((NC, blk, CNT_W), lambda i: (0, i, 0)),
          pl.BlockSpec((blk, d), lambda i: (i, 0)),
          pl.BlockSpec((d, d), lambda i: (0, 0)),
          pl.BlockSpec((1, d), lambda i: (0, 0)),
          pl.BlockSpec((d, d), lambda i: (0, 0)),
          pl.BlockSpec((d, out), lambda i: (0, 0)),
          pl.BlockSpec((1, out), lambda i: (0, 0)),
      ],
      out_specs=pl.BlockSpec((blk, out), lambda i: (i, 0)),
      out_shape=jax.ShapeDtypeStruct((n, out), jnp.float32),
  )(p, cnt, h, wlT, bl, wrT, woT, bo)


def kernel(x, edge_index, W1l, b1l, W1r, W2l, b2l, W2r, W3l, b3l, W3r,
           Wout, bout):
  n, d = x.shape
  e = edge_index.shape[1]
  stripe = NS * ZR
  n_pad = -(-n // stripe) * stripe
  x_pad = jnp.pad(x, ((0, n_pad - n), (0, 0)))
  src3 = edge_index[0].reshape(NW, -1, CH)
  dst3 = edge_index[1].reshape(NW, -1, CH)

  agg = _make_sc_agg(n_pad, d, e)
  counts = _make_sc_counts(n_pad, e)

  c1 = counts(dst3)
  p1 = agg(x_pad, src3, dst3)
  h1 = _tc_layer(p1, c1, x_pad, W1l.T, b1l.reshape(1, -1), W1r.T)
  p2 = agg(h1, src3, dst3)
  h2 = _tc_layer(p2, c1, h1, W2l.T, b2l.reshape(1, -1), W2r.T)
  p3 = agg(h2, src3, dst3)
  out = _tc_final(p3, c1, h2, W3l.T, b3l.reshape(1, -1), W3r.T,
                  Wout.T, bout.reshape(1, -1))
  return out[:n]


# speedup vs baseline: 9.6804x; 1.2320x over previous
import jax
import jax.numpy as jnp
from jax import lax
from jax.experimental import pallas as pl
from jax.experimental.pallas import tpu as pltpu
from jax.experimental.pallas import tpu_sc as plsc

NC = 2
NS = 16
NW = NC * NS
CH = 125
CNT_W = 128
ZR = 128


def _make_sc_agg(n, d, e):
  k_chunks = e // (NW * CH)
  assert k_chunks * NW * CH == e
  rows_per_sub = n // NS
  assert rows_per_sub * NS == n and rows_per_sub % ZR == 0
  n_dma = rows_per_sub // ZR

  ib = 16
  assert k_chunks % ib == 0 and ib % 2 == 0
  mesh = plsc.VectorSubcoreMesh(core_axis_name="c", subcore_axis_name="s")
  scratch = [
      pltpu.VMEM((ib, CH), jnp.int32),
      pltpu.VMEM((ib, CH), jnp.int32),
      pltpu.VMEM((CH, d), jnp.float32),
      pltpu.VMEM((CH, d), jnp.float32),
      pltpu.VMEM_SHARED((n, d), jnp.float32),
      pltpu.SemaphoreType.DMA,
      pltpu.SemaphoreType.DMA,
  ]

  def body(h_hbm, src_hbm, dst_hbm, out_hbm, src_v, dst_v, rows_a, rows_b,
           acc_sh, sem_a, sem_b):
    cid = lax.axis_index("c")
    sid = lax.axis_index("s")
    wid = cid * NS + sid
    base = sid * rows_per_sub

    @pl.loop(0, CH)
    def _(r):
      @pl.loop(0, d, step=16)
      def _(c):
        rows_a[pl.ds(r, 1), pl.ds(c, 16)] = jnp.zeros((1, 16), jnp.float32)

    nz_full, nz_rem = rows_per_sub // CH, rows_per_sub % CH
    for t in range(nz_full):
      pltpu.sync_copy(rows_a, acc_sh.at[pl.ds(base + t * CH, CH)])
    if nz_rem:
      pltpu.sync_copy(rows_a.at[pl.ds(0, nz_rem)],
                      acc_sh.at[pl.ds(base + nz_full * CH, nz_rem)])
    plsc.subcore_barrier()

    @pl.loop(0, k_chunks // ib)
    def _(g):
      pltpu.sync_copy(src_hbm.at[wid].at[pl.ds(g * ib, ib)], src_v)
      pltpu.sync_copy(dst_hbm.at[wid].at[pl.ds(g * ib, ib)], dst_v)

      pltpu.async_copy(h_hbm.at[src_v.at[0]], rows_a, sem_a)

      @pl.loop(0, ib // 2)
      def _(m):
        j = m * 2
        pltpu.make_async_copy(h_hbm.at[src_v.at[j]], rows_a, sem_a).wait()
        pltpu.async_copy(h_hbm.at[src_v.at[j + 1]], rows_b, sem_b)
        pltpu.sync_copy(rows_a, acc_sh.at[dst_v.at[j]], add=True)
        pltpu.make_async_copy(h_hbm.at[src_v.at[j + 1]], rows_b, sem_b).wait()

        @pl.when(m < ib // 2 - 1)
        def _():
          pltpu.async_copy(h_hbm.at[src_v.at[j + 2]], rows_a, sem_a)

        pltpu.sync_copy(rows_b, acc_sh.at[dst_v.at[j + 1]], add=True)

    plsc.subcore_barrier()

    for t in range(n_dma):
      sl = pl.ds(base + t * ZR, ZR)
      pltpu.sync_copy(acc_sh.at[sl], out_hbm.at[cid].at[sl])

  return pl.kernel(
      body,
      out_type=jax.ShapeDtypeStruct((NC, n, d), jnp.float32),
      mesh=mesh,
      scratch_types=scratch,
  )


def _make_sc_counts(n, e, cw=CNT_W):
  k_chunks = e // (NW * CH)
  rows_per_sub = n // NS
  n_dma = rows_per_sub // ZR

  mesh = plsc.VectorSubcoreMesh(core_axis_name="c", subcore_axis_name="s")
  scratch = [
      pltpu.VMEM((k_chunks, CH), jnp.int32),
      pltpu.VMEM((CH, cw), jnp.float32),
      pltpu.VMEM((ZR, cw), jnp.float32),
      pltpu.VMEM_SHARED((n, cw), jnp.float32),
  ]

  def body(dst_hbm, cnt_hbm, dst_v, ones_v, zcnt_v, cnt_sh):
    cid = lax.axis_index("c")
    sid = lax.axis_index("s")
    wid = cid * NS + sid
    base = sid * rows_per_sub

    @pl.loop(0, ZR)
    def _(r):
      @pl.loop(0, cw, step=16)
      def _(c):
        zcnt_v[pl.ds(r, 1), pl.ds(c, 16)] = jnp.zeros((1, 16), jnp.float32)

    @pl.loop(0, CH)
    def _(r):
      @pl.loop(0, cw, step=16)
      def _(c):
        ones_v[pl.ds(r, 1), pl.ds(c, 16)] = jnp.ones((1, 16), jnp.float32)

    for t in range(n_dma):
      pltpu.sync_copy(zcnt_v, cnt_sh.at[pl.ds(base + t * ZR, ZR)])
    plsc.subcore_barrier()

    pltpu.sync_copy(dst_hbm.at[wid], dst_v)

    @pl.loop(0, k_chunks)
    def _(j):
      pltpu.sync_copy(ones_v, cnt_sh.at[dst_v.at[j]], add=True)

    plsc.subcore_barrier()

    for t in range(n_dma):
      sl = pl.ds(base + t * ZR, ZR)
      pltpu.sync_copy(cnt_sh.at[sl], cnt_hbm.at[cid].at[sl])

  return pl.kernel(
      body,
      out_type=jax.ShapeDtypeStruct((NC, n, cw), jnp.float32),
      mesh=mesh,
      scratch_types=scratch,
  )


def _tc_layer_body(p_ref, c_ref, h_ref, wl_ref, bl_ref, wr_ref, o_ref):
  s = p_ref[0] + p_ref[1]
  c = c_ref[0, :, 0:1] + c_ref[1, :, 0:1]
  mean = s / jnp.maximum(c, 1.0)
  z = jnp.dot(mean, wl_ref[...], preferred_element_type=jnp.float32)
  z = z + bl_ref[...]
  z = z + jnp.dot(h_ref[...], wr_ref[...], preferred_element_type=jnp.float32)
  o_ref[...] = jnp.maximum(z, 0.0)


def _tc_final_body(p_ref, c_ref, h_ref, wl_ref, bl_ref, wr_ref, wo_ref,
                   bo_ref, o_ref):
  s = p_ref[0] + p_ref[1]
  c = c_ref[0, :, 0:1] + c_ref[1, :, 0:1]
  mean = s / jnp.maximum(c, 1.0)
  z = jnp.dot(mean, wl_ref[...], preferred_element_type=jnp.float32)
  z = z + bl_ref[...]
  z = z + jnp.dot(h_ref[...], wr_ref[...], preferred_element_type=jnp.float32)
  h3 = jnp.maximum(z, 0.0)
  o_ref[...] = jnp.dot(h3, wo_ref[...],
                       preferred_element_type=jnp.float32) + bo_ref[...]


def _tc_layer(p, cnt, h, wlT, bl, wrT, blk=1024):
  n, d = h.shape
  return pl.pallas_call(
      _tc_layer_body,
      grid=(n // blk,),
      in_specs=[
          pl.BlockSpec((NC, blk, d), lambda i: (0, i, 0)),
          pl.BlockSpec((NC, blk, CNT_W), lambda i: (0, i, 0)),
          pl.BlockSpec((blk, d), lambda i: (i, 0)),
          pl.BlockSpec((d, d), lambda i: (0, 0)),
          pl.BlockSpec((1, d), lambda i: (0, 0)),
          pl.BlockSpec((d, d), lambda i: (0, 0)),
      ],
      out_specs=pl.BlockSpec((blk, d), lambda i: (i, 0)),
      out_shape=jax.ShapeDtypeStruct((n, d), jnp.float32),
  )(p, cnt, h, wlT, bl, wrT)


def _tc_final(p, cnt, h, wlT, bl, wrT, woT, bo, blk=1024):
  n, d = h.shape
  out = woT.shape[1]
  return pl.pallas_call(
      _tc_final_body,
      grid=(n // blk,),
      in_specs=[
          pl.BlockSpec((NC, blk, d), lambda i: (0, i, 0)),
          pl.BlockSpec((NC, blk, CNT_W), lambda i: (0, i, 0)),
          pl.BlockSpec((blk, d), lambda i: (i, 0)),
          pl.BlockSpec((d, d), lambda i: (0, 0)),
          pl.BlockSpec((1, d), lambda i: (0, 0)),
          pl.BlockSpec((d, d), lambda i: (0, 0)),
          pl.BlockSpec((d, out), lambda i: (0, 0)),
          pl.BlockSpec((1, out), lambda i: (0, 0)),
      ],
      out_specs=pl.BlockSpec((blk, out), lambda i: (i, 0)),
      out_shape=jax.ShapeDtypeStruct((n, out), jnp.float32),
  )(p, cnt, h, wlT, bl, wrT, woT, bo)


def kernel(x, edge_index, W1l, b1l, W1r, W2l, b2l, W2r, W3l, b3l, W3r,
           Wout, bout):
  n, d = x.shape
  e = edge_index.shape[1]
  stripe = NS * ZR
  n_pad = -(-n // stripe) * stripe
  x_pad = jnp.pad(x, ((0, n_pad - n), (0, 0)))
  src3 = edge_index[0].reshape(NW, -1, CH)
  dst3 = edge_index[1].reshape(NW, -1, CH)

  agg = _make_sc_agg(n_pad, d, e)
  counts = _make_sc_counts(n_pad, e)

  c1 = counts(dst3)
  p1 = agg(x_pad, src3, dst3)
  h1 = _tc_layer(p1, c1, x_pad, W1l.T, b1l.reshape(1, -1), W1r.T)
  p2 = agg(h1, src3, dst3)
  h2 = _tc_layer(p2, c1, h1, W2l.T, b2l.reshape(1, -1), W2r.T)
  p3 = agg(h2, src3, dst3)
  out = _tc_final(p3, c1, h2, W3l.T, b3l.reshape(1, -1), W3r.T,
                  Wout.T, bout.reshape(1, -1))
  return out[:n]


# speedup vs baseline: 11.4160x; 1.1793x over previous
import jax
import jax.numpy as jnp
from jax import lax
from jax.experimental import pallas as pl
from jax.experimental.pallas import tpu as pltpu
from jax.experimental.pallas import tpu_sc as plsc

NC = 2
NS = 16
NW = NC * NS
CH = 50
CNT_W = 128


def _zero_fill(zbuf, shared, base, nrows):
  zr = zbuf.shape[0]
  full, rem = nrows // zr, nrows % zr
  for t in range(full):
    pltpu.sync_copy(zbuf, shared.at[pl.ds(base + t * zr, zr)])
  if rem:
    pltpu.sync_copy(zbuf.at[pl.ds(0, rem)],
                    shared.at[pl.ds(base + full * zr, rem)])


def _make_sc_agg(n, d, e):
  k_chunks = e // (NW * CH)
  assert k_chunks * NW * CH == e
  rows_per_sub = n // NS
  assert rows_per_sub * NS == n and rows_per_sub % 8 == 0

  ib = 40
  la = 2
  ibx = ib + 8
  assert k_chunks % ib == 0 and ib % 4 == 0
  n_pieces = k_chunks // ib
  mesh = plsc.VectorSubcoreMesh(core_axis_name="c", subcore_axis_name="s")
  scratch = [
      pltpu.VMEM((2 * ibx, CH), jnp.int32),
      pltpu.VMEM((2 * ib, CH), jnp.int32),
      pltpu.VMEM((4 * CH, d), jnp.float32),
      pltpu.VMEM_SHARED((n, d), jnp.float32),
      pltpu.SemaphoreType.DMA,
      pltpu.SemaphoreType.DMA,
      pltpu.SemaphoreType.DMA,
      pltpu.SemaphoreType.DMA,
      pltpu.SemaphoreType.DMA,
      pltpu.SemaphoreType.DMA,
      pltpu.SemaphoreType.DMA,
      pltpu.SemaphoreType.DMA,
      pltpu.SemaphoreType.DMA,
  ]

  def body(h_hbm, src_hbm, dst_hbm, out_hbm, src_all, dst_all, ring,
           acc_sh, g0, g1, g2, g3, s0, s1, s2, s3, sem_i):
    rows = [ring.at[pl.ds(b * CH, CH)] for b in range(4)]
    gsem = [g0, g1, g2, g3]
    ssem = [s0, s1, s2, s3]
    srcs = [src_all.at[pl.ds(0, ibx)], src_all.at[pl.ds(ibx, ibx)]]
    dsts = [dst_all.at[pl.ds(0, ib)], dst_all.at[pl.ds(ib, ib)]]
    cid = lax.axis_index("c")
    sid = lax.axis_index("s")
    wid = cid * NS + sid
    base = sid * rows_per_sub

    @pl.loop(0, CH)
    def _(r):
      @pl.loop(0, d, step=16)
      def _(c):
        ring[pl.ds(r, 1), pl.ds(c, 16)] = jnp.zeros((1, 16), jnp.float32)

    _zero_fill(rows[0], acc_sh, base, rows_per_sub)
    plsc.subcore_barrier()

    pltpu.sync_copy(src_hbm.at[wid].at[pl.ds(0, ibx)], srcs[0])
    pltpu.sync_copy(dst_hbm.at[wid].at[pl.ds(0, ib)], dsts[0])
    for b in range(la):
      pltpu.async_copy(h_hbm.at[srcs[0].at[b]], rows[b], gsem[b])

    for g in range(n_pieces):
      sv, dv = srcs[g % 2], dsts[g % 2]
      if g + 1 < n_pieces:
        nxt = (g + 1) * ib
        nrows = ibx if g + 1 < n_pieces - 1 else ib
        pltpu.async_copy(src_hbm.at[wid].at[pl.ds(nxt, nrows)],
                         srcs[(g + 1) % 2].at[pl.ds(0, nrows)], sem_i)
        pltpu.async_copy(dst_hbm.at[wid].at[pl.ds(nxt, ib)],
                         dsts[(g + 1) % 2], sem_i)

      @pl.loop(0, ib // 4)
      def _(m):
        for b in range(4):
          jl = m * 4 + b
          jg = g * ib + jl
          bl = (b + la) % 4
          @pl.when(jg + la < k_chunks)
          def _():
            @pl.when(jg + la >= 4)
            def _():
              pltpu.make_async_copy(rows[bl], acc_sh.at[dv.at[0]],
                                    ssem[bl]).wait()
            pltpu.async_copy(h_hbm.at[sv.at[jl + la]], rows[bl], gsem[bl])

          pltpu.make_async_copy(h_hbm.at[sv.at[jl]], rows[b],
                                gsem[b]).wait()
          pltpu.async_copy(rows[b], acc_sh.at[dv.at[jl]], ssem[b],
                           add=True)

      if g + 1 < n_pieces:
        nrows = ibx if g + 1 < n_pieces - 1 else ib
        pltpu.make_async_copy(src_hbm.at[wid].at[pl.ds((g + 1) * ib, nrows)],
                              srcs[(g + 1) % 2].at[pl.ds(0, nrows)],
                              sem_i).wait()
        pltpu.make_async_copy(dst_hbm.at[wid].at[pl.ds((g + 1) * ib, ib)],
                              dsts[(g + 1) % 2], sem_i).wait()

    for b in range(4):
      pltpu.make_async_copy(rows[b], acc_sh.at[dsts[(n_pieces - 1) % 2].at[0]],
                            ssem[b]).wait()

    plsc.subcore_barrier()

    sl = pl.ds(base, rows_per_sub)
    pltpu.sync_copy(acc_sh.at[sl], out_hbm.at[cid].at[sl])

  return pl.kernel(
      body,
      out_type=jax.ShapeDtypeStruct((NC, n, d), jnp.float32),
      mesh=mesh,
      scratch_types=scratch,
  )


def _make_sc_counts(n, e, cw=CNT_W):
  k_chunks = e // (NW * CH)
  rows_per_sub = n // NS

  mesh = plsc.VectorSubcoreMesh(core_axis_name="c", subcore_axis_name="s")
  scratch = [
      pltpu.VMEM((k_chunks, CH), jnp.int32),
      pltpu.VMEM((CH, cw), jnp.float32),
      pltpu.VMEM((CH, cw), jnp.float32),
      pltpu.VMEM_SHARED((n, cw), jnp.float32),
  ]

  def body(dst_hbm, cnt_hbm, dst_v, ones_v, zcnt_v, cnt_sh):
    cid = lax.axis_index("c")
    sid = lax.axis_index("s")
    wid = cid * NS + sid
    base = sid * rows_per_sub

    @pl.loop(0, CH)
    def _(r):
      @pl.loop(0, cw, step=16)
      def _(c):
        zcnt_v[pl.ds(r, 1), pl.ds(c, 16)] = jnp.zeros((1, 16), jnp.float32)
        ones_v[pl.ds(r, 1), pl.ds(c, 16)] = jnp.ones((1, 16), jnp.float32)

    _zero_fill(zcnt_v, cnt_sh, base, rows_per_sub)
    plsc.subcore_barrier()

    pltpu.sync_copy(dst_hbm.at[wid], dst_v)

    @pl.loop(0, k_chunks)
    def _(j):
      pltpu.sync_copy(ones_v, cnt_sh.at[dst_v.at[j]], add=True)

    plsc.subcore_barrier()

    sl = pl.ds(base, rows_per_sub)
    pltpu.sync_copy(cnt_sh.at[sl], cnt_hbm.at[cid].at[sl])

  return pl.kernel(
      body,
      out_type=jax.ShapeDtypeStruct((NC, n, cw), jnp.float32),
      mesh=mesh,
      scratch_types=scratch,
  )


def _tc_layer_body(p_ref, c_ref, h_ref, wl_ref, bl_ref, wr_ref, o_ref):
  s = p_ref[0] + p_ref[1]
  c = c_ref[0, :, 0:1] + c_ref[1, :, 0:1]
  mean = s / jnp.maximum(c, 1.0)
  z = jnp.dot(mean, wl_ref[...], preferred_element_type=jnp.float32)
  z = z + bl_ref[...]
  z = z + jnp.dot(h_ref[...], wr_ref[...], preferred_element_type=jnp.float32)
  o_ref[...] = jnp.maximum(z, 0.0)


def _tc_final_body(p_ref, c_ref, h_ref, wl_ref, bl_ref, wr_ref, wo_ref,
                   bo_ref, o_ref):
  s = p_ref[0] + p_ref[1]
  c = c_ref[0, :, 0:1] + c_ref[1, :, 0:1]
  mean = s / jnp.maximum(c, 1.0)
  z = jnp.dot(mean, wl_ref[...], preferred_element_type=jnp.float32)
  z = z + bl_ref[...]
  z = z + jnp.dot(h_ref[...], wr_ref[...], preferred_element_type=jnp.float32)
  h3 = jnp.maximum(z, 0.0)
  o_ref[...] = jnp.dot(h3, wo_ref[...],
                       preferred_element_type=jnp.float32) + bo_ref[...]


def _tc_layer(p, cnt, h, wlT, bl, wrT):
  n, d = h.shape
  blk = n // 8
  return pl.pallas_call(
      _tc_layer_body,
      grid=(n // blk,),
      in_specs=[
          pl.BlockSpec((NC, blk, d), lambda i: (0, i, 0)),
          pl.BlockSpec((NC, blk, CNT_W), lambda i: (0, i, 0)),
          pl.BlockSpec((blk, d), lambda i: (i, 0)),
          pl.BlockSpec((d, d), lambda i: (0, 0)),
          pl.BlockSpec((1, d), lambda i: (0, 0)),
          pl.BlockSpec((d, d), lambda i: (0, 0)),
      ],
      out_specs=pl.BlockSpec((blk, d), lambda i: (i, 0)),
      out_shape=jax.ShapeDtypeStruct((n, d), jnp.float32),
  )(p, cnt, h, wlT, bl, wrT)


def _tc_final(p, cnt, h, wlT, bl, wrT, woT, bo):
  n, d = h.shape
  blk = n // 8
  out = woT.shape[1]
  return pl.pallas_call(
      _tc_final_body,
      grid=(n // blk,),
      in_specs=[
          pl.BlockSpec((NC, blk, d), lambda i: (0, i, 0)),
          pl.BlockSpec((NC, blk, CNT_W), lambda i: (0, i, 0)),
          pl.BlockSpec((blk, d), lambda i: (i, 0)),
          pl.BlockSpec((d, d), lambda i: (0, 0)),
          pl.BlockSpec((1, d), lambda i: (0, 0)),
          pl.BlockSpec((d, d), lambda i: (0, 0)),
          pl.BlockSpec((d, out), lambda i: (0, 0)),
          pl.BlockSpec((1, out), lambda i: (0, 0)),
      ],
      out_specs=pl.BlockSpec((blk, out), lambda i: (i, 0)),
      out_shape=jax.ShapeDtypeStruct((n, out), jnp.float32),
  )(p, cnt, h, wlT, bl, wrT, woT, bo)


def kernel(x, edge_index, W1l, b1l, W1r, W2l, b2l, W2r, W3l, b3l, W3r,
           Wout, bout):
  n, d = x.shape
  e = edge_index.shape[1]
  rps = -(-(-(-n // NS)) // 8) * 8
  n_pad = NS * rps
  x_pad = jnp.pad(x, ((0, n_pad - n), (0, 0)))
  src3 = edge_index[0].reshape(NW, -1, CH)
  dst3 = edge_index[1].reshape(NW, -1, CH)

  agg = _make_sc_agg(n_pad, d, e)
  counts = _make_sc_counts(n_pad, e)

  c1 = counts(dst3)
  p1 = agg(x_pad, src3, dst3)
  h1 = _tc_layer(p1, c1, x_pad, W1l.T, b1l.reshape(1, -1), W1r.T)
  p2 = agg(h1, src3, dst3)
  h2 = _tc_layer(p2, c1, h1, W2l.T, b2l.reshape(1, -1), W2r.T)
  p3 = agg(h2, src3, dst3)
  out = _tc_final(p3, c1, h2, W3l.T, b3l.reshape(1, -1), W3r.T,
                  Wout.T, bout.reshape(1, -1))
  return out[:n]


# speedup vs baseline: 13.2791x; 1.1632x over previous
import dataclasses

import jax
import jax.numpy as jnp
from jax import lax
from jax.experimental import pallas as pl
from jax.experimental.pallas import tpu as pltpu
from jax.experimental.pallas import tpu_sc as plsc

NC = 2
NS = 16
NW = NC * NS
CH = 50
CNT_W = 16

_CP_NO_LAYOUT = pltpu.CompilerParams()
if "needs_layout_passes" in pltpu.CompilerParams.__dataclass_fields__:
  _CP_NO_LAYOUT = dataclasses.replace(_CP_NO_LAYOUT,
                                      needs_layout_passes=False)


def _zero_fill(zbuf, shared, base, nrows):
  zr = zbuf.shape[0]
  full, rem = nrows // zr, nrows % zr
  for t in range(full):
    pltpu.sync_copy(zbuf, shared.at[pl.ds(base + t * zr, zr)])
  if rem:
    pltpu.sync_copy(zbuf.at[pl.ds(0, rem)],
                    shared.at[pl.ds(base + full * zr, rem)])


def _make_sc_agg(n, d, e):
  k_chunks = e // (NW * CH)
  assert k_chunks * NW * CH == e
  rows_per_sub = n // NS
  assert rows_per_sub * NS == n and rows_per_sub % 8 == 0

  ib = 40
  la = 2
  ibx = ib + 8
  assert k_chunks % ib == 0 and ib % 4 == 0
  n_pieces = k_chunks // ib
  mesh = plsc.VectorSubcoreMesh(core_axis_name="c", subcore_axis_name="s")
  scratch = [
      pltpu.VMEM((2 * ibx, CH), jnp.int32),
      pltpu.VMEM((2 * ib, CH), jnp.int32),
      pltpu.VMEM((4 * CH, d), jnp.float32),
      pltpu.VMEM_SHARED((n, d), jnp.float32),
      pltpu.SemaphoreType.DMA,
      pltpu.SemaphoreType.DMA,
      pltpu.SemaphoreType.DMA,
      pltpu.SemaphoreType.DMA,
      pltpu.SemaphoreType.DMA,
      pltpu.SemaphoreType.DMA,
      pltpu.SemaphoreType.DMA,
      pltpu.SemaphoreType.DMA,
      pltpu.SemaphoreType.DMA,
  ]

  def body(h_hbm, src_hbm, dst_hbm, out_hbm, src_all, dst_all, ring,
           acc_sh, g0, g1, g2, g3, s0, s1, s2, s3, sem_i):
    rows = [ring.at[pl.ds(b * CH, CH)] for b in range(4)]
    gsem = [g0, g1, g2, g3]
    ssem = [s0, s1, s2, s3]
    srcs = [src_all.at[pl.ds(0, ibx)], src_all.at[pl.ds(ibx, ibx)]]
    dsts = [dst_all.at[pl.ds(0, ib)], dst_all.at[pl.ds(ib, ib)]]
    cid = lax.axis_index("c")
    sid = lax.axis_index("s")
    wid = cid * NS + sid
    base = sid * rows_per_sub

    @pl.loop(0, CH)
    def _(r):
      @pl.loop(0, d, step=16)
      def _(c):
        ring[pl.ds(r, 1), pl.ds(c, 16)] = jnp.zeros((1, 16), jnp.float32)

    _zero_fill(rows[0], acc_sh, base, rows_per_sub)
    plsc.subcore_barrier()

    pltpu.sync_copy(src_hbm.at[wid].at[pl.ds(0, ibx)], srcs[0])
    pltpu.sync_copy(dst_hbm.at[wid].at[pl.ds(0, ib)], dsts[0])
    for b in range(la):
      pltpu.async_copy(h_hbm.at[srcs[0].at[b]], rows[b], gsem[b])

    for g in range(n_pieces):
      sv, dv = srcs[g % 2], dsts[g % 2]
      if g + 1 < n_pieces:
        nxt = (g + 1) * ib
        nrows = ibx if g + 1 < n_pieces - 1 else ib
        pltpu.async_copy(src_hbm.at[wid].at[pl.ds(nxt, nrows)],
                         srcs[(g + 1) % 2].at[pl.ds(0, nrows)], sem_i)
        pltpu.async_copy(dst_hbm.at[wid].at[pl.ds(nxt, ib)],
                         dsts[(g + 1) % 2], sem_i)

      @pl.loop(0, ib // 4)
      def _(m):
        for b in range(4):
          jl = m * 4 + b
          jg = g * ib + jl
          bl = (b + la) % 4
          @pl.when(jg + la < k_chunks)
          def _():
            @pl.when(jg + la >= 4)
            def _():
              pltpu.make_async_copy(rows[bl], acc_sh.at[dv.at[0]],
                                    ssem[bl]).wait()
            pltpu.async_copy(h_hbm.at[sv.at[jl + la]], rows[bl], gsem[bl])

          pltpu.make_async_copy(h_hbm.at[sv.at[jl]], rows[b],
                                gsem[b]).wait()
          pltpu.async_copy(rows[b], acc_sh.at[dv.at[jl]], ssem[b],
                           add=True)

      if g + 1 < n_pieces:
        nrows = ibx if g + 1 < n_pieces - 1 else ib
        pltpu.make_async_copy(src_hbm.at[wid].at[pl.ds((g + 1) * ib, nrows)],
                              srcs[(g + 1) % 2].at[pl.ds(0, nrows)],
                              sem_i).wait()
        pltpu.make_async_copy(dst_hbm.at[wid].at[pl.ds((g + 1) * ib, ib)],
                              dsts[(g + 1) % 2], sem_i).wait()

    for b in range(4):
      pltpu.make_async_copy(rows[b], acc_sh.at[dsts[(n_pieces - 1) % 2].at[0]],
                            ssem[b]).wait()

    plsc.subcore_barrier()

    sl = pl.ds(base, rows_per_sub)
    pltpu.sync_copy(acc_sh.at[sl], out_hbm.at[cid].at[sl])

  return pl.kernel(
      body,
      out_type=jax.ShapeDtypeStruct((NC, n, d), jnp.float32),
      mesh=mesh,
      scratch_types=scratch,
  )


def _make_sc_counts(nb, e):
  ew = e // NW
  assert ew * NW == e and ew % 16 == 0
  slab = nb // NS
  assert slab % 16 == 0

  mesh = plsc.VectorSubcoreMesh(core_axis_name="c", subcore_axis_name="s")
  scratch = [
      pltpu.VMEM((ew,), jnp.int32),
      pltpu.VMEM((nb,), jnp.float32),
      pltpu.VMEM((slab * 16,), jnp.float32),
      pltpu.VMEM_SHARED((NS, nb), jnp.float32),
  ]

  def body(dst_hbm, cnt_hbm, dst_v, priv_v, stage_v, pub_sh):
    cid = lax.axis_index("c")
    sid = lax.axis_index("s")
    wid = cid * NS + sid
    base = sid * slab

    pltpu.sync_copy(dst_hbm.at[wid], dst_v)

    @pl.loop(0, nb, step=16)
    def _(i):
      priv_v[pl.ds(i, 16)] = jnp.zeros((16,), jnp.float32)

    ones = jnp.ones((16,), jnp.float32)

    @pl.loop(0, ew, step=16)
    def _(j):
      plsc.addupdate_scatter(priv_v, [dst_v[pl.ds(j, 16)]], ones)

    pltpu.sync_copy(priv_v, pub_sh.at[sid])
    plsc.subcore_barrier()

    pltpu.sync_copy(pub_sh.at[0].at[pl.ds(base, slab)],
                    priv_v.at[pl.ds(0, slab)])
    for r in range(1, NS):
      pltpu.sync_copy(pub_sh.at[r].at[pl.ds(base, slab)],
                      priv_v.at[pl.ds(slab, slab)])

      @pl.loop(0, slab, step=16)
      def _(v):
        priv_v[pl.ds(v, 16)] += priv_v[pl.ds(slab + v, 16)]

    @pl.loop(0, slab, step=16)
    def _(v):
      c = priv_v[pl.ds(v, 16)]
      for rr in range(16):
        stage_v[pl.ds((v + rr) * 16, 16)] = jnp.take(
            c, jnp.full((16,), rr, jnp.int32))

    pltpu.sync_copy(stage_v, cnt_hbm.at[cid].at[pl.ds(base * 16, slab * 16)])

  return pl.kernel(
      body,
      out_type=jax.ShapeDtypeStruct((NC, nb * 16), jnp.float32),
      mesh=mesh,
      compiler_params=_CP_NO_LAYOUT,
      scratch_types=scratch,
  )


def _tc_layer_body(p_ref, c_ref, h_ref, wl_ref, bl_ref, wr_ref, o_ref):
  s = p_ref[0] + p_ref[1]
  c = c_ref[0, :, 0:1] + c_ref[1, :, 0:1]
  mean = s / jnp.maximum(c, 1.0)
  z = jnp.dot(mean, wl_ref[...], preferred_element_type=jnp.float32)
  z = z + bl_ref[...]
  z = z + jnp.dot(h_ref[...], wr_ref[...], preferred_element_type=jnp.float32)
  o_ref[...] = jnp.maximum(z, 0.0)


def _tc_final_body(p_ref, c_ref, h_ref, wl_ref, bl_ref, wr_ref, wo_ref,
                   bo_ref, o_ref):
  s = p_ref[0] + p_ref[1]
  c = c_ref[0, :, 0:1] + c_ref[1, :, 0:1]
  mean = s / jnp.maximum(c, 1.0)
  z = jnp.dot(mean, wl_ref[...], preferred_element_type=jnp.float32)
  z = z + bl_ref[...]
  z = z + jnp.dot(h_ref[...], wr_ref[...], preferred_element_type=jnp.float32)
  h3 = jnp.maximum(z, 0.0)
  o_ref[...] = jnp.dot(h3, wo_ref[...],
                       preferred_element_type=jnp.float32) + bo_ref[...]


def _tc_layer(p, cnt, h, wlT, bl, wrT):
  n, d = h.shape
  blk = n // 8
  return pl.pallas_call(
      _tc_layer_body,
      grid=(n // blk,),
      in_specs=[
          pl.BlockSpec((NC, blk, d), lambda i: (0, i, 0)),
          pl.BlockSpec((NC, blk, CNT_W), lambda i: (0, i, 0)),
          pl.BlockSpec((blk, d), lambda i: (i, 0)),
          pl.BlockSpec((d, d), lambda i: (0, 0)),
          pl.BlockSpec((1, d), lambda i: (0, 0)),
          pl.BlockSpec((d, d), lambda i: (0, 0)),
      ],
      out_specs=pl.BlockSpec((blk, d), lambda i: (i, 0)),
      out_shape=jax.ShapeDtypeStruct((n, d), jnp.float32),
  )(p, cnt, h, wlT, bl, wrT)


def _tc_final(p, cnt, h, wlT, bl, wrT, woT, bo):
  n, d = h.shape
  blk = n // 8
  out = woT.shape[1]
  return pl.pallas_call(
      _tc_final_body,
      grid=(n // blk,),
      in_specs=[
          pl.BlockSpec((NC, blk, d), lambda i: (0, i, 0)),
          pl.BlockSpec((NC, blk, CNT_W), lambda i: (0, i, 0)),
          pl.BlockSpec((blk, d), lambda i: (i, 0)),
          pl.BlockSpec((d, d), lambda i: (0, 0)),
          pl.BlockSpec((1, d), lambda i: (0, 0)),
          pl.BlockSpec((d, d), lambda i: (0, 0)),
          pl.BlockSpec((d, out), lambda i: (0, 0)),
          pl.BlockSpec((1, out), lambda i: (0, 0)),
      ],
      out_specs=pl.BlockSpec((blk, out), lambda i: (i, 0)),
      out_shape=jax.ShapeDtypeStruct((n, out), jnp.float32),
  )(p, cnt, h, wlT, bl, wrT, woT, bo)


def kernel(x, edge_index, W1l, b1l, W1r, W2l, b2l, W2r, W3l, b3l, W3r,
           Wout, bout):
  n, d = x.shape
  e = edge_index.shape[1]
  rps = -(-(-(-n // NS)) // 8) * 8
  n_pad = NS * rps
  x_pad = jnp.pad(x, ((0, n_pad - n), (0, 0)))
  src3 = edge_index[0].reshape(NW, -1, CH)
  dst3 = edge_index[1].reshape(NW, -1, CH)

  agg = _make_sc_agg(n_pad, d, e)
  nb = -(-n_pad // (16 * NS)) * (16 * NS)
  counts = _make_sc_counts(nb, e)

  dst_flat = edge_index[1].reshape(NW, -1)
  c1 = counts(dst_flat).reshape(NC, nb, CNT_W)
  p1 = agg(x_pad, src3, dst3)
  h1 = _tc_layer(p1, c1, x_pad, W1l.T, b1l.reshape(1, -1), W1r.T)
  p2 = agg(h1, src3, dst3)
  h2 = _tc_layer(p2, c1, h1, W2l.T, b2l.reshape(1, -1), W2r.T)
  p3 = agg(h2, src3, dst3)
  out = _tc_final(p3, c1, h2, W3l.T, b3l.reshape(1, -1), W3r.T,
                  Wout.T, bout.reshape(1, -1))
  return out[:n]


# speedup vs baseline: 13.3226x; 1.0033x over previous
import dataclasses

import jax
import jax.numpy as jnp
from jax import lax
from jax.experimental import pallas as pl
from jax.experimental.pallas import tpu as pltpu
from jax.experimental.pallas import tpu_sc as plsc

NC = 2
NS = 16
NW = NC * NS
CH = 50
CNT_W = 16

_CP_NO_LAYOUT = pltpu.CompilerParams()
if "needs_layout_passes" in pltpu.CompilerParams.__dataclass_fields__:
  _CP_NO_LAYOUT = dataclasses.replace(_CP_NO_LAYOUT,
                                      needs_layout_passes=False)


def _zero_fill(zbuf, shared, base, nrows):
  zr = zbuf.shape[0]
  full, rem = nrows // zr, nrows % zr
  for t in range(full):
    pltpu.sync_copy(zbuf, shared.at[pl.ds(base + t * zr, zr)])
  if rem:
    pltpu.sync_copy(zbuf.at[pl.ds(0, rem)],
                    shared.at[pl.ds(base + full * zr, rem)])


def _make_sc_agg(n, d, e):
  k_chunks = e // (NW * CH)
  assert k_chunks * NW * CH == e
  rows_per_sub = n // NS
  assert rows_per_sub * NS == n and rows_per_sub % 8 == 0

  ib = 40
  la = 2
  ibx = ib + 8
  assert k_chunks % ib == 0 and ib % 4 == 0
  n_pieces = k_chunks // ib
  mesh = plsc.VectorSubcoreMesh(core_axis_name="c", subcore_axis_name="s")
  scratch = [
      pltpu.VMEM((2 * ibx, CH), jnp.int32),
      pltpu.VMEM((2 * ib, CH), jnp.int32),
      pltpu.VMEM((4 * CH, d), jnp.float32),
      pltpu.VMEM_SHARED((n, d), jnp.float32),
      pltpu.SemaphoreType.DMA,
      pltpu.SemaphoreType.DMA,
      pltpu.SemaphoreType.DMA,
      pltpu.SemaphoreType.DMA,
      pltpu.SemaphoreType.DMA,
      pltpu.SemaphoreType.DMA,
      pltpu.SemaphoreType.DMA,
      pltpu.SemaphoreType.DMA,
      pltpu.SemaphoreType.DMA,
  ]

  def body(h_hbm, src_hbm, dst_hbm, out_hbm, src_all, dst_all, ring,
           acc_sh, g0, g1, g2, g3, s0, s1, s2, s3, sem_i):
    rows = [ring.at[pl.ds(b * CH, CH)] for b in range(4)]
    gsem = [g0, g1, g2, g3]
    ssem = [s0, s1, s2, s3]
    srcs = [src_all.at[pl.ds(0, ibx)], src_all.at[pl.ds(ibx, ibx)]]
    dsts = [dst_all.at[pl.ds(0, ib)], dst_all.at[pl.ds(ib, ib)]]
    cid = lax.axis_index("c")
    sid = lax.axis_index("s")
    wid = cid * NS + sid
    base = sid * rows_per_sub

    @pl.loop(0, CH)
    def _(r):
      @pl.loop(0, d, step=16)
      def _(c):
        ring[pl.ds(r, 1), pl.ds(c, 16)] = jnp.zeros((1, 16), jnp.float32)

    _zero_fill(rows[0], acc_sh, base, rows_per_sub)
    plsc.subcore_barrier()

    pltpu.sync_copy(src_hbm.at[wid].at[pl.ds(0, ibx)], srcs[0])
    pltpu.sync_copy(dst_hbm.at[wid].at[pl.ds(0, ib)], dsts[0])
    for b in range(la):
      pltpu.async_copy(h_hbm.at[srcs[0].at[b]], rows[b], gsem[b])

    nb_dma = CH * d * 4

    def chunk_body(sv, dv, jl, b, ssem_wait=True, issue=True):
      bl = (b + la) % 4
      if issue:
        if ssem_wait:
          pltpu.make_async_copy(rows[bl], acc_sh.at[dv.at[0]],
                                ssem[bl]).wait()
        pltpu.async_copy(h_hbm.at[sv.at[jl + la]], rows[bl], gsem[bl])
      pltpu.make_async_copy(h_hbm.at[sv.at[0]], rows[b], gsem[b]).wait()
      pltpu.async_copy(rows[b], acc_sh.at[dv.at[jl]], ssem[b], add=True)

    for g in range(n_pieces):
      sv, dv = srcs[g % 2], dsts[g % 2]

      for b in range(4):
        chunk_body(sv, dv, b, b, ssem_wait=(g > 0 or b >= 4 - la))

      if g + 1 < n_pieces:
        nxt = (g + 1) * ib
        nrows = ibx if g + 1 < n_pieces - 1 else ib
        pltpu.async_copy(src_hbm.at[wid].at[pl.ds(nxt, nrows)],
                         srcs[(g + 1) % 2].at[pl.ds(0, nrows)], sem_i)
        pltpu.async_copy(dst_hbm.at[wid].at[pl.ds(nxt, ib)],
                         dsts[(g + 1) % 2], sem_i)

      m_hi = ib // 4 - (1 if g == n_pieces - 1 else 0)

      @pl.loop(1, m_hi)
      def _(m):
        for b in range(4):
          chunk_body(sv, dv, m * 4 + b, b)

      if g == n_pieces - 1:
        for b in range(4):
          chunk_body(sv, dv, ib - 4 + b, b, issue=(b < 4 - la))

      if g + 1 < n_pieces:
        nrows = ibx if g + 1 < n_pieces - 1 else ib
        pltpu.make_async_copy(src_hbm.at[wid].at[pl.ds((g + 1) * ib, nrows)],
                              srcs[(g + 1) % 2].at[pl.ds(0, nrows)],
                              sem_i).wait()
        pltpu.make_async_copy(dst_hbm.at[wid].at[pl.ds((g + 1) * ib, ib)],
                              dsts[(g + 1) % 2], sem_i).wait()

    for b in range(4):
      pltpu.make_async_copy(rows[b], acc_sh.at[dsts[(n_pieces - 1) % 2].at[0]],
                            ssem[b]).wait()

    plsc.subcore_barrier()

    sl = pl.ds(base, rows_per_sub)
    pltpu.sync_copy(acc_sh.at[sl], out_hbm.at[cid].at[sl])

  return pl.kernel(
      body,
      out_type=jax.ShapeDtypeStruct((NC, n, d), jnp.float32),
      mesh=mesh,
      scratch_types=scratch,
  )


def _make_sc_counts(nb, e):
  ew = e // NW
  assert ew * NW == e and ew % 16 == 0
  slab = nb // NS
  assert slab % 16 == 0

  mesh = plsc.VectorSubcoreMesh(core_axis_name="c", subcore_axis_name="s")
  scratch = [
      pltpu.VMEM((ew,), jnp.int32),
      pltpu.VMEM((nb,), jnp.float32),
      pltpu.VMEM((slab * 16,), jnp.float32),
      pltpu.VMEM_SHARED((NS, nb), jnp.float32),
  ]

  def body(dst_hbm, cnt_hbm, dst_v, priv_v, stage_v, pub_sh):
    cid = lax.axis_index("c")
    sid = lax.axis_index("s")
    wid = cid * NS + sid
    base = sid * slab

    pltpu.sync_copy(dst_hbm.at[wid], dst_v)

    @pl.loop(0, nb, step=16)
    def _(i):
      priv_v[pl.ds(i, 16)] = jnp.zeros((16,), jnp.float32)

    ones = jnp.ones((16,), jnp.float32)

    @pl.loop(0, ew, step=16)
    def _(j):
      plsc.addupdate_scatter(priv_v, [dst_v[pl.ds(j, 16)]], ones)

    pltpu.sync_copy(priv_v, pub_sh.at[sid])
    plsc.subcore_barrier()

    pltpu.sync_copy(pub_sh.at[0].at[pl.ds(base, slab)],
                    priv_v.at[pl.ds(0, slab)])
    for r in range(1, NS):
      pltpu.sync_copy(pub_sh.at[r].at[pl.ds(base, slab)],
                      priv_v.at[pl.ds(slab, slab)])

      @pl.loop(0, slab, step=16)
      def _(v):
        priv_v[pl.ds(v, 16)] += priv_v[pl.ds(slab + v, 16)]

    @pl.loop(0, slab, step=16)
    def _(v):
      c = priv_v[pl.ds(v, 16)]
      for rr in range(16):
        stage_v[pl.ds((v + rr) * 16, 16)] = jnp.take(
            c, jnp.full((16,), rr, jnp.int32))

    pltpu.sync_copy(stage_v, cnt_hbm.at[cid].at[pl.ds(base * 16, slab * 16)])

  return pl.kernel(
      body,
      out_type=jax.ShapeDtypeStruct((NC, nb * 16), jnp.float32),
      mesh=mesh,
      compiler_params=_CP_NO_LAYOUT,
      scratch_types=scratch,
  )


def _tc_layer_body(p_ref, c_ref, h_ref, wl_ref, bl_ref, wr_ref, o_ref):
  s = p_ref[0] + p_ref[1]
  c = c_ref[0, :, 0:1] + c_ref[1, :, 0:1]
  mean = s / jnp.maximum(c, 1.0)
  z = jnp.dot(mean, wl_ref[...], preferred_element_type=jnp.float32)
  z = z + bl_ref[...]
  z = z + jnp.dot(h_ref[...], wr_ref[...], preferred_element_type=jnp.float32)
  o_ref[...] = jnp.maximum(z, 0.0)


def _tc_final_body(p_ref, c_ref, h_ref, wl_ref, bl_ref, wr_ref, wo_ref,
                   bo_ref, o_ref):
  s = p_ref[0] + p_ref[1]
  c = c_ref[0, :, 0:1] + c_ref[1, :, 0:1]
  mean = s / jnp.maximum(c, 1.0)
  z = jnp.dot(mean, wl_ref[...], preferred_element_type=jnp.float32)
  z = z + bl_ref[...]
  z = z + jnp.dot(h_ref[...], wr_ref[...], preferred_element_type=jnp.float32)
  h3 = jnp.maximum(z, 0.0)
  o_ref[...] = jnp.dot(h3, wo_ref[...],
                       preferred_element_type=jnp.float32) + bo_ref[...]


def _tc_layer(p, cnt, h, wlT, bl, wrT):
  n, d = h.shape
  blk = n // 8
  return pl.pallas_call(
      _tc_layer_body,
      grid=(n // blk,),
      in_specs=[
          pl.BlockSpec((NC, blk, d), lambda i: (0, i, 0)),
          pl.BlockSpec((NC, blk, CNT_W), lambda i: (0, i, 0)),
          pl.BlockSpec((blk, d), lambda i: (i, 0)),
          pl.BlockSpec((d, d), lambda i: (0, 0)),
          pl.BlockSpec((1, d), lambda i: (0, 0)),
          pl.BlockSpec((d, d), lambda i: (0, 0)),
      ],
      out_specs=pl.BlockSpec((blk, d), lambda i: (i, 0)),
      out_shape=jax.ShapeDtypeStruct((n, d), jnp.float32),
  )(p, cnt, h, wlT, bl, wrT)


def _tc_final(p, cnt, h, wlT, bl, wrT, woT, bo):
  n, d = h.shape
  blk = n // 8
  out = woT.shape[1]
  return pl.pallas_call(
      _tc_final_body,
      grid=(n // blk,),
      in_specs=[
          pl.BlockSpec((NC, blk, d), lambda i: (0, i, 0)),
          pl.BlockSpec((NC, blk, CNT_W), lambda i: (0, i, 0)),
          pl.BlockSpec((blk, d), lambda i: (i, 0)),
          pl.BlockSpec((d, d), lambda i: (0, 0)),
          pl.BlockSpec((1, d), lambda i: (0, 0)),
          pl.BlockSpec((d, d), lambda i: (0, 0)),
          pl.BlockSpec((d, out), lambda i: (0, 0)),
          pl.BlockSpec((1, out), lambda i: (0, 0)),
      ],
      out_specs=pl.BlockSpec((blk, out), lambda i: (i, 0)),
      out_shape=jax.ShapeDtypeStruct((n, out), jnp.float32),
  )(p, cnt, h, wlT, bl, wrT, woT, bo)


def kernel(x, edge_index, W1l, b1l, W1r, W2l, b2l, W2r, W3l, b3l, W3r,
           Wout, bout):
  n, d = x.shape
  e = edge_index.shape[1]
  rps = -(-(-(-n // NS)) // 8) * 8
  n_pad = NS * rps
  x_pad = jnp.pad(x, ((0, n_pad - n), (0, 0)))
  src3 = edge_index[0].reshape(NW, -1, CH)
  dst3 = edge_index[1].reshape(NW, -1, CH)

  agg = _make_sc_agg(n_pad, d, e)
  nb = -(-n_pad // (16 * NS)) * (16 * NS)
  counts = _make_sc_counts(nb, e)

  dst_flat = edge_index[1].reshape(NW, -1)
  c1 = counts(dst_flat).reshape(NC, nb, CNT_W)
  p1 = agg(x_pad, src3, dst3)
  h1 = _tc_layer(p1, c1, x_pad, W1l.T, b1l.reshape(1, -1), W1r.T)
  p2 = agg(h1, src3, dst3)
  h2 = _tc_layer(p2, c1, h1, W2l.T, b2l.reshape(1, -1), W2r.T)
  p3 = agg(h2, src3, dst3)
  out = _tc_final(p3, c1, h2, W3l.T, b3l.reshape(1, -1), W3r.T,
                  Wout.T, bout.reshape(1, -1))
  return out[:n]


# speedup vs baseline: 14.1172x; 1.0596x over previous
import dataclasses

import jax
import jax.numpy as jnp
from jax import lax
from jax.experimental import pallas as pl
from jax.experimental.pallas import tpu as pltpu
from jax.experimental.pallas import tpu_sc as plsc

NC = 2
NS = 16
NW = NC * NS
CH = 50
CNT_W = 16

_CP_NO_LAYOUT = pltpu.CompilerParams()
if "needs_layout_passes" in pltpu.CompilerParams.__dataclass_fields__:
  _CP_NO_LAYOUT = dataclasses.replace(_CP_NO_LAYOUT,
                                      needs_layout_passes=False)


def _zero_fill(zbuf, shared, base, nrows):
  zr = zbuf.shape[0]
  full, rem = nrows // zr, nrows % zr
  for t in range(full):
    pltpu.sync_copy(zbuf, shared.at[pl.ds(base + t * zr, zr)])
  if rem:
    pltpu.sync_copy(zbuf.at[pl.ds(0, rem)],
                    shared.at[pl.ds(base + full * zr, rem)])


def _make_sc_agg(n, d, e):
  k_chunks = e // (NW * CH)
  assert k_chunks * NW * CH == e
  rows_per_sub = n // NS
  assert rows_per_sub * NS == n and rows_per_sub % 8 == 0

  ib = 40
  R = 4
  la = 3
  ibx = ib + 8
  assert k_chunks % ib == 0 and ib % R == 0
  n_pieces = k_chunks // ib
  mesh = plsc.VectorSubcoreMesh(core_axis_name="c", subcore_axis_name="s")
  scratch = [
      pltpu.VMEM((2 * ibx, CH), jnp.int32),
      pltpu.VMEM((2 * ib, CH), jnp.int32),
      pltpu.VMEM((R * CH, d), jnp.float32),
      pltpu.VMEM_SHARED((n, d), jnp.float32),
  ] + [pltpu.SemaphoreType.DMA] * (2 * R + 1)

  def body(h_hbm, src_hbm, dst_hbm, out_hbm, src_all, dst_all, ring,
           acc_sh, *sems):
    gsem = list(sems[:R])
    ssem = list(sems[R:2 * R])
    sem_i = sems[2 * R]
    rows = [ring.at[pl.ds(b * CH, CH)] for b in range(R)]
    srcs = [src_all.at[pl.ds(0, ibx)], src_all.at[pl.ds(ibx, ibx)]]
    dsts = [dst_all.at[pl.ds(0, ib)], dst_all.at[pl.ds(ib, ib)]]
    cid = lax.axis_index("c")
    sid = lax.axis_index("s")
    wid = cid * NS + sid
    base = sid * rows_per_sub

    @pl.loop(0, CH)
    def _(r):
      @pl.loop(0, d, step=16)
      def _(c):
        ring[pl.ds(r, 1), pl.ds(c, 16)] = jnp.zeros((1, 16), jnp.float32)

    _zero_fill(rows[0], acc_sh, base, rows_per_sub)
    plsc.subcore_barrier()

    pltpu.sync_copy(src_hbm.at[wid].at[pl.ds(0, ibx)], srcs[0])
    pltpu.sync_copy(dst_hbm.at[wid].at[pl.ds(0, ib)], dsts[0])
    for b in range(la):
      pltpu.async_copy(h_hbm.at[srcs[0].at[b]], rows[b], gsem[b])

    nb_dma = CH * d * 4

    def chunk_body(sv, dv, jl, b, ssem_wait=True, issue=True):
      bl = (b + la) % R
      if issue:
        if ssem_wait:
          pltpu.make_async_copy(rows[bl], acc_sh.at[dv.at[0]],
                                ssem[bl]).wait()
        pltpu.async_copy(h_hbm.at[sv.at[jl + la]], rows[bl], gsem[bl])
      pltpu.make_async_copy(h_hbm.at[sv.at[0]], rows[b], gsem[b]).wait()
      pltpu.async_copy(rows[b], acc_sh.at[dv.at[jl]], ssem[b], add=True)

    for g in range(n_pieces):
      sv, dv = srcs[g % 2], dsts[g % 2]

      for b in range(R):
        chunk_body(sv, dv, b, b, ssem_wait=(g > 0 or b >= R - la))

      if g + 1 < n_pieces:
        nxt = (g + 1) * ib
        nrows = ibx if g + 1 < n_pieces - 1 else ib
        pltpu.async_copy(src_hbm.at[wid].at[pl.ds(nxt, nrows)],
                         srcs[(g + 1) % 2].at[pl.ds(0, nrows)], sem_i)
        pltpu.async_copy(dst_hbm.at[wid].at[pl.ds(nxt, ib)],
                         dsts[(g + 1) % 2], sem_i)

      m_hi = ib // R - (1 if g == n_pieces - 1 else 0)

      @pl.loop(1, m_hi)
      def _(m):
        for b in range(R):
          chunk_body(sv, dv, m * R + b, b)

      if g == n_pieces - 1:
        for b in range(R):
          chunk_body(sv, dv, ib - R + b, b, issue=(b < R - la))

      if g + 1 < n_pieces:
        nrows = ibx if g + 1 < n_pieces - 1 else ib
        pltpu.make_async_copy(src_hbm.at[wid].at[pl.ds((g + 1) * ib, nrows)],
                              srcs[(g + 1) % 2].at[pl.ds(0, nrows)],
                              sem_i).wait()
        pltpu.make_async_copy(dst_hbm.at[wid].at[pl.ds((g + 1) * ib, ib)],
                              dsts[(g + 1) % 2], sem_i).wait()

    for b in range(R):
      pltpu.make_async_copy(rows[b], acc_sh.at[dsts[(n_pieces - 1) % 2].at[0]],
                            ssem[b]).wait()

    plsc.subcore_barrier()

    sl = pl.ds(base, rows_per_sub)
    pltpu.sync_copy(acc_sh.at[sl], out_hbm.at[cid].at[sl])

  return pl.kernel(
      body,
      out_type=jax.ShapeDtypeStruct((NC, n, d), jnp.float32),
      mesh=mesh,
      scratch_types=scratch,
  )


def _make_sc_counts(nb, e):
  ew = e // NW
  assert ew * NW == e and ew % 16 == 0
  slab = nb // NS
  assert slab % 16 == 0

  mesh = plsc.VectorSubcoreMesh(core_axis_name="c", subcore_axis_name="s")
  scratch = [
      pltpu.VMEM((ew,), jnp.int32),
      pltpu.VMEM((nb,), jnp.float32),
      pltpu.VMEM((slab * 16,), jnp.float32),
      pltpu.VMEM_SHARED((NS, nb), jnp.float32),
  ]

  def body(dst_hbm, cnt_hbm, dst_v, priv_v, stage_v, pub_sh):
    cid = lax.axis_index("c")
    sid = lax.axis_index("s")
    wid = cid * NS + sid
    base = sid * slab

    pltpu.sync_copy(dst_hbm.at[wid], dst_v)

    @pl.loop(0, nb, step=16)
    def _(i):
      priv_v[pl.ds(i, 16)] = jnp.zeros((16,), jnp.float32)

    ones = jnp.ones((16,), jnp.float32)

    @pl.loop(0, ew, step=16)
    def _(j):
      plsc.addupdate_scatter(priv_v, [dst_v[pl.ds(j, 16)]], ones)

    pltpu.sync_copy(priv_v, pub_sh.at[sid])
    plsc.subcore_barrier()

    pltpu.sync_copy(pub_sh.at[0].at[pl.ds(base, slab)],
                    priv_v.at[pl.ds(0, slab)])
    for r in range(1, NS):
      pltpu.sync_copy(pub_sh.at[r].at[pl.ds(base, slab)],
                      priv_v.at[pl.ds(slab, slab)])

      @pl.loop(0, slab, step=16)
      def _(v):
        priv_v[pl.ds(v, 16)] += priv_v[pl.ds(slab + v, 16)]

    @pl.loop(0, slab, step=16)
    def _(v):
      c = priv_v[pl.ds(v, 16)]
      for rr in range(16):
        stage_v[pl.ds((v + rr) * 16, 16)] = jnp.take(
            c, jnp.full((16,), rr, jnp.int32))

    pltpu.sync_copy(stage_v, cnt_hbm.at[cid].at[pl.ds(base * 16, slab * 16)])

  return pl.kernel(
      body,
      out_type=jax.ShapeDtypeStruct((NC, nb * 16), jnp.float32),
      mesh=mesh,
      compiler_params=_CP_NO_LAYOUT,
      scratch_types=scratch,
  )


def _tc_layer_body(p_ref, c_ref, h_ref, wl_ref, bl_ref, wr_ref, o_ref):
  s = p_ref[0] + p_ref[1]
  c = c_ref[0, :, 0:1] + c_ref[1, :, 0:1]
  mean = s / jnp.maximum(c, 1.0)
  z = jnp.dot(mean, wl_ref[...], preferred_element_type=jnp.float32)
  z = z + bl_ref[...]
  z = z + jnp.dot(h_ref[...], wr_ref[...], preferred_element_type=jnp.float32)
  o_ref[...] = jnp.maximum(z, 0.0)


def _tc_final_body(p_ref, c_ref, h_ref, wl_ref, bl_ref, wr_ref, wo_ref,
                   bo_ref, o_ref):
  s = p_ref[0] + p_ref[1]
  c = c_ref[0, :, 0:1] + c_ref[1, :, 0:1]
  mean = s / jnp.maximum(c, 1.0)
  z = jnp.dot(mean, wl_ref[...], preferred_element_type=jnp.float32)
  z = z + bl_ref[...]
  z = z + jnp.dot(h_ref[...], wr_ref[...], preferred_element_type=jnp.float32)
  h3 = jnp.maximum(z, 0.0)
  o_ref[...] = jnp.dot(h3, wo_ref[...],
                       preferred_element_type=jnp.float32) + bo_ref[...]


def _tc_layer(p, cnt, h, wlT, bl, wrT):
  n, d = h.shape
  blk = n // 8
  return pl.pallas_call(
      _tc_layer_body,
      grid=(n // blk,),
      in_specs=[
          pl.BlockSpec((NC, blk, d), lambda i: (0, i, 0)),
          pl.BlockSpec((NC, blk, CNT_W), lambda i: (0, i, 0)),
          pl.BlockSpec((blk, d), lambda i: (i, 0)),
          pl.BlockSpec((d, d), lambda i: (0, 0)),
          pl.BlockSpec((1, d), lambda i: (0, 0)),
          pl.BlockSpec((d, d), lambda i: (0, 0)),
      ],
      out_specs=pl.BlockSpec((blk, d), lambda i: (i, 0)),
      out_shape=jax.ShapeDtypeStruct((n, d), jnp.float32),
  )(p, cnt, h, wlT, bl, wrT)


def _tc_final(p, cnt, h, wlT, bl, wrT, woT, bo):
  n, d = h.shape
  blk = n // 8
  out = woT.shape[1]
  return pl.pallas_call(
      _tc_final_body,
      grid=(n // blk,),
      in_specs=[
          pl.BlockSpec((NC, blk, d), lambda i: (0, i, 0)),
          pl.BlockSpec((NC, blk, CNT_W), lambda i: (0, i, 0)),
          pl.BlockSpec((blk, d), lambda i: (i, 0)),
          pl.BlockSpec((d, d), lambda i: (0, 0)),
          pl.BlockSpec((1, d), lambda i: (0, 0)),
          pl.BlockSpec((d, d), lambda i: (0, 0)),
          pl.BlockSpec((d, out), lambda i: (0, 0)),
          pl.BlockSpec((1, out), lambda i: (0, 0)),
      ],
      out_specs=pl.BlockSpec((blk, out), lambda i: (i, 0)),
      out_shape=jax.ShapeDtypeStruct((n, out), jnp.float32),
  )(p, cnt, h, wlT, bl, wrT, woT, bo)


def kernel(x, edge_index, W1l, b1l, W1r, W2l, b2l, W2r, W3l, b3l, W3r,
           Wout, bout):
  n, d = x.shape
  e = edge_index.shape[1]
  rps = -(-(-(-n // NS)) // 8) * 8
  n_pad = NS * rps
  x_pad = jnp.pad(x, ((0, n_pad - n), (0, 0)))
  src3 = edge_index[0].reshape(NW, -1, CH)
  dst3 = edge_index[1].reshape(NW, -1, CH)

  agg = _make_sc_agg(n_pad, d, e)
  nb = -(-n_pad // (16 * NS)) * (16 * NS)
  counts = _make_sc_counts(nb, e)

  dst_flat = edge_index[1].reshape(NW, -1)
  c1 = counts(dst_flat).reshape(NC, nb, CNT_W)
  p1 = agg(x_pad, src3, dst3)
  h1 = _tc_layer(p1, c1, x_pad, W1l.T, b1l.reshape(1, -1), W1r.T)
  p2 = agg(h1, src3, dst3)
  h2 = _tc_layer(p2, c1, h1, W2l.T, b2l.reshape(1, -1), W2r.T)
  p3 = agg(h2, src3, dst3)
  out = _tc_final(p3, c1, h2, W3l.T, b3l.reshape(1, -1), W3r.T,
                  Wout.T, bout.reshape(1, -1))
  return out[:n]


# speedup vs baseline: 14.2778x; 1.0114x over previous
import dataclasses

import jax
import jax.numpy as jnp
from jax import lax
from jax.experimental import pallas as pl
from jax.experimental.pallas import tpu as pltpu
from jax.experimental.pallas import tpu_sc as plsc

NC = 2
NS = 16
NW = NC * NS
CH = 50
CNT_W = 16

_CP_NO_LAYOUT = pltpu.CompilerParams()
if "needs_layout_passes" in pltpu.CompilerParams.__dataclass_fields__:
  _CP_NO_LAYOUT = dataclasses.replace(_CP_NO_LAYOUT,
                                      needs_layout_passes=False)


def _zero_fill(zbuf, shared, base, nrows):
  zr = zbuf.shape[0]
  full, rem = nrows // zr, nrows % zr
  for t in range(full):
    pltpu.sync_copy(zbuf, shared.at[pl.ds(base + t * zr, zr)])
  if rem:
    pltpu.sync_copy(zbuf.at[pl.ds(0, rem)],
                    shared.at[pl.ds(base + full * zr, rem)])


def _make_sc_agg(n, d, e, n_src=None):
  del n_src
  k_chunks = e // (NW * CH)
  assert k_chunks * NW * CH == e
  rows_per_sub = n // NS
  assert rows_per_sub * NS == n and rows_per_sub % 8 == 0

  ib = 40
  R = 4
  la = 3
  ibx = ib + 8
  assert k_chunks % ib == 0 and ib % R == 0
  n_pieces = k_chunks // ib
  mesh = plsc.VectorSubcoreMesh(core_axis_name="c", subcore_axis_name="s")
  scratch = [
      pltpu.VMEM((2 * ibx, CH), jnp.int32),
      pltpu.VMEM((2 * ib, CH), jnp.int32),
      pltpu.VMEM((R * CH, d), jnp.float32),
      pltpu.VMEM_SHARED((n, d), jnp.float32),
  ] + [pltpu.SemaphoreType.DMA] * (2 * R + 1)

  def body(h_hbm, src_hbm, dst_hbm, out_hbm, src_all, dst_all, ring,
           acc_sh, *sems):
    gsem = list(sems[:R])
    ssem = list(sems[R:2 * R])
    sem_i = sems[2 * R]
    rows = [ring.at[pl.ds(b * CH, CH)] for b in range(R)]
    srcs = [src_all.at[pl.ds(0, ibx)], src_all.at[pl.ds(ibx, ibx)]]
    dsts = [dst_all.at[pl.ds(0, ib)], dst_all.at[pl.ds(ib, ib)]]
    cid = lax.axis_index("c")
    sid = lax.axis_index("s")
    wid = cid * NS + sid
    base = sid * rows_per_sub

    @pl.loop(0, CH)
    def _(r):
      @pl.loop(0, d, step=16)
      def _(c):
        ring[pl.ds(r, 1), pl.ds(c, 16)] = jnp.zeros((1, 16), jnp.float32)

    _zero_fill(rows[0], acc_sh, base, rows_per_sub)
    plsc.subcore_barrier()

    pltpu.sync_copy(src_hbm.at[wid].at[pl.ds(0, ibx)], srcs[0])
    pltpu.sync_copy(dst_hbm.at[wid].at[pl.ds(0, ib)], dsts[0])
    for b in range(la):
      pltpu.async_copy(h_hbm.at[srcs[0].at[b]], rows[b], gsem[b])

    nb_dma = CH * d * 4

    def chunk_body(sv, dv, jl, b, ssem_wait=True, issue=True):
      bl = (b + la) % R
      if issue:
        if ssem_wait:
          pltpu.make_async_copy(rows[bl], acc_sh.at[dv.at[0]],
                                ssem[bl]).wait()
        pltpu.async_copy(h_hbm.at[sv.at[jl + la]], rows[bl], gsem[bl])
      pltpu.make_async_copy(h_hbm.at[sv.at[0]], rows[b], gsem[b]).wait()
      pltpu.async_copy(rows[b], acc_sh.at[dv.at[jl]], ssem[b], add=True)

    for g in range(n_pieces):
      sv, dv = srcs[g % 2], dsts[g % 2]

      for b in range(R):
        chunk_body(sv, dv, b, b, ssem_wait=(g > 0 or b >= R - la))

      if g + 1 < n_pieces:
        nxt = (g + 1) * ib
        nrows = ibx if g + 1 < n_pieces - 1 else ib
        pltpu.async_copy(src_hbm.at[wid].at[pl.ds(nxt, nrows)],
                         srcs[(g + 1) % 2].at[pl.ds(0, nrows)], sem_i)
        pltpu.async_copy(dst_hbm.at[wid].at[pl.ds(nxt, ib)],
                         dsts[(g + 1) % 2], sem_i)

      m_hi = ib // R - (1 if g == n_pieces - 1 else 0)

      @pl.loop(1, m_hi)
      def _(m):
        for b in range(R):
          chunk_body(sv, dv, m * R + b, b)

      if g == n_pieces - 1:
        for b in range(R):
          chunk_body(sv, dv, ib - R + b, b, issue=(b < R - la))

      if g + 1 < n_pieces:
        nrows = ibx if g + 1 < n_pieces - 1 else ib
        pltpu.make_async_copy(src_hbm.at[wid].at[pl.ds((g + 1) * ib, nrows)],
                              srcs[(g + 1) % 2].at[pl.ds(0, nrows)],
                              sem_i).wait()
        pltpu.make_async_copy(dst_hbm.at[wid].at[pl.ds((g + 1) * ib, ib)],
                              dsts[(g + 1) % 2], sem_i).wait()

    for b in range(R):
      pltpu.make_async_copy(rows[b], acc_sh.at[dsts[(n_pieces - 1) % 2].at[0]],
                            ssem[b]).wait()

    plsc.subcore_barrier()

    sl = pl.ds(base, rows_per_sub)
    pltpu.sync_copy(acc_sh.at[sl], out_hbm.at[cid].at[sl])

  return pl.kernel(
      body,
      out_type=jax.ShapeDtypeStruct((NC, n, d), jnp.float32),
      mesh=mesh,
      scratch_types=scratch,
  )


def _make_sc_counts(nb, e):
  ew = e // NW
  assert ew * NW == e and ew % 16 == 0
  slab = nb // NS
  assert slab % 16 == 0

  mesh = plsc.VectorSubcoreMesh(core_axis_name="c", subcore_axis_name="s")
  scratch = [
      pltpu.VMEM((ew,), jnp.int32),
      pltpu.VMEM((nb,), jnp.float32),
      pltpu.VMEM((slab * 16,), jnp.float32),
      pltpu.VMEM_SHARED((NS, nb), jnp.float32),
  ]

  def body(dst_hbm, cnt_hbm, dst_v, priv_v, stage_v, pub_sh):
    cid = lax.axis_index("c")
    sid = lax.axis_index("s")
    wid = cid * NS + sid
    base = sid * slab

    pltpu.sync_copy(dst_hbm.at[wid], dst_v)

    @pl.loop(0, nb, step=16)
    def _(i):
      priv_v[pl.ds(i, 16)] = jnp.zeros((16,), jnp.float32)

    ones = jnp.ones((16,), jnp.float32)

    @pl.loop(0, ew, step=16)
    def _(j):
      plsc.addupdate_scatter(priv_v, [dst_v[pl.ds(j, 16)]], ones)

    pltpu.sync_copy(priv_v, pub_sh.at[sid])
    plsc.subcore_barrier()

    pltpu.sync_copy(pub_sh.at[0].at[pl.ds(base, slab)],
                    priv_v.at[pl.ds(0, slab)])
    for r in range(1, NS):
      pltpu.sync_copy(pub_sh.at[r].at[pl.ds(base, slab)],
                      priv_v.at[pl.ds(slab, slab)])

      @pl.loop(0, slab, step=16)
      def _(v):
        priv_v[pl.ds(v, 16)] += priv_v[pl.ds(slab + v, 16)]

    @pl.loop(0, slab, step=16)
    def _(v):
      c = priv_v[pl.ds(v, 16)]
      for rr in range(16):
        stage_v[pl.ds((v + rr) * 16, 16)] = jnp.take(
            c, jnp.full((16,), rr, jnp.int32))

    pltpu.sync_copy(stage_v, cnt_hbm.at[cid].at[pl.ds(base * 16, slab * 16)])

  return pl.kernel(
      body,
      out_type=jax.ShapeDtypeStruct((NC, nb * 16), jnp.float32),
      mesh=mesh,
      compiler_params=_CP_NO_LAYOUT,
      scratch_types=scratch,
  )


def _tc_right_body(h_ref, wr_ref, bl_ref, o_ref):
  o_ref[...] = jnp.dot(h_ref[...], wr_ref[...],
                       preferred_element_type=jnp.float32) + bl_ref[...]


def _tc_right(h, wrT, bl, blk=2000):
  n, d = h.shape
  return pl.pallas_call(
      _tc_right_body,
      grid=(n // blk,),
      in_specs=[
          pl.BlockSpec((blk, d), lambda i: (i, 0)),
          pl.BlockSpec((d, d), lambda i: (0, 0)),
          pl.BlockSpec((1, d), lambda i: (0, 0)),
      ],
      out_specs=pl.BlockSpec((blk, d), lambda i: (i, 0)),
      out_shape=jax.ShapeDtypeStruct((n, d), jnp.float32),
  )(h, wrT, bl)


def _tc_combine_body(p_ref, c_ref, r_ref, wl_ref, o_ref):
  s = p_ref[0] + p_ref[1]
  c = c_ref[0, :, 0:1] + c_ref[1, :, 0:1]
  mean = s / jnp.maximum(c, 1.0)
  z = jnp.dot(mean, wl_ref[...], preferred_element_type=jnp.float32)
  o_ref[...] = jnp.maximum(z + r_ref[...], 0.0)


def _tc_combine_final_body(p_ref, c_ref, r_ref, wl_ref, wo_ref, bo_ref,
                           o_ref):
  s = p_ref[0] + p_ref[1]
  c = c_ref[0, :, 0:1] + c_ref[1, :, 0:1]
  mean = s / jnp.maximum(c, 1.0)
  z = jnp.dot(mean, wl_ref[...], preferred_element_type=jnp.float32)
  h3 = jnp.maximum(z + r_ref[...], 0.0)
  o_ref[...] = jnp.dot(h3, wo_ref[...],
                       preferred_element_type=jnp.float32) + bo_ref[...]


def _tc_combine(p, cnt, r, wlT, blk=2000):
  n, d = r.shape
  return pl.pallas_call(
      _tc_combine_body,
      grid=(n // blk,),
      in_specs=[
          pl.BlockSpec((NC, blk, d), lambda i: (0, i, 0)),
          pl.BlockSpec((NC, blk, CNT_W), lambda i: (0, i, 0)),
          pl.BlockSpec((blk, d), lambda i: (i, 0)),
          pl.BlockSpec((d, d), lambda i: (0, 0)),
      ],
      out_specs=pl.BlockSpec((blk, d), lambda i: (i, 0)),
      out_shape=jax.ShapeDtypeStruct((n, d), jnp.float32),
  )(p, cnt, r, wlT)


def _tc_combine_final(p, cnt, r, wlT, woT, bo, blk=2000):
  n, d = r.shape
  out = woT.shape[1]
  return pl.pallas_call(
      _tc_combine_final_body,
      grid=(n // blk,),
      in_specs=[
          pl.BlockSpec((NC, blk, d), lambda i: (0, i, 0)),
          pl.BlockSpec((NC, blk, CNT_W), lambda i: (0, i, 0)),
          pl.BlockSpec((blk, d), lambda i: (i, 0)),
          pl.BlockSpec((d, d), lambda i: (0, 0)),
          pl.BlockSpec((d, out), lambda i: (0, 0)),
          pl.BlockSpec((1, out), lambda i: (0, 0)),
      ],
      out_specs=pl.BlockSpec((blk, out), lambda i: (i, 0)),
      out_shape=jax.ShapeDtypeStruct((n, out), jnp.float32),
  )(p, cnt, r, wlT, woT, bo)


def kernel(x, edge_index, W1l, b1l, W1r, W2l, b2l, W2r, W3l, b3l, W3r,
           Wout, bout):
  n, d = x.shape
  e = edge_index.shape[1]
  rps = -(-(-(-n // NS)) // 8) * 8
  n_pad = NS * rps
  src3 = edge_index[0].reshape(NW, -1, CH)
  dst3 = edge_index[1].reshape(NW, -1, CH)

  agg = _make_sc_agg(n_pad, d, e, n)
  nb = -(-n_pad // (16 * NS)) * (16 * NS)
  counts = _make_sc_counts(nb, e)

  dst_flat = edge_index[1].reshape(NW, -1)
  c1 = counts(dst_flat).reshape(NC, nb, CNT_W)
  r1 = _tc_right(x, W1r.T, b1l.reshape(1, -1))
  p1 = agg(x, src3, dst3)
  h1 = _tc_combine(p1, c1, r1, W1l.T)
  r2 = _tc_right(h1, W2r.T, b2l.reshape(1, -1))
  p2 = agg(h1, src3, dst3)
  h2 = _tc_combine(p2, c1, r2, W2l.T)
  r3 = _tc_right(h2, W3r.T, b3l.reshape(1, -1))
  p3 = agg(h2, src3, dst3)
  return _tc_combine_final(p3, c1, r3, W3l.T, Wout.T, bout.reshape(1, -1))
